# baseline (device time: 175410 ns/iter reference)
import jax
import jax.numpy as jnp
from jax import lax
from jax.experimental import pallas as pl
from jax.experimental.pallas import tpu as pltpu

N_DEV = 8
M = 2048
N = 2048
CHUNK = M // N_DEV
HALF = N // 2
Q = 8
PC = CHUNK // Q


def _gelu(y):
    c = 0.7978845608028654
    return 0.5 * y * (1.0 + jnp.tanh(c * (y + 0.044715 * y * y * y)))


def kernel(x, w_mat):
    def body(x_ref, w_ref, out_ref, cw_ref, ccw_ref,
             cw_ssem, cw_rsem, ccw_ssem, ccw_rsem):
        i = lax.axis_index("i")
        left = lax.rem(i + N_DEV - 1, N_DEV)
        right = lax.rem(i + 1, N_DEV)

        barrier = pltpu.get_barrier_semaphore()
        pl.semaphore_signal(barrier, inc=1, device_id=(left,),
                            device_id_type=pl.DeviceIdType.MESH)
        pl.semaphore_signal(barrier, inc=1, device_id=(right,),
                            device_id_type=pl.DeviceIdType.MESH)
        pl.semaphore_wait(barrier, 2)

        def chunk_cw(c):
            xc = x_ref[pl.ds(c * CHUNK, CHUNK), :]
            return jnp.dot(xc, w_ref[:, 0:HALF],
                           preferred_element_type=jnp.float32)

        def chunk_ccw(c):
            xc = x_ref[pl.ds(c * CHUNK, CHUNK), :]
            return jnp.dot(xc, w_ref[:, HALF:N],
                           preferred_element_type=jnp.float32)

        def mk(buf, ssem, rsem, k_src, k_dst, q, dev):
            return pltpu.make_async_remote_copy(
                src_ref=buf.at[k_src, pl.ds(q * PC, PC)],
                dst_ref=buf.at[k_dst, pl.ds(q * PC, PC)],
                send_sem=ssem.at[k_src, q],
                recv_sem=rsem.at[k_dst, q],
                device_id=(dev,), device_id_type=pl.DeviceIdType.MESH)

        def mk_cw(k_src, k_dst, q):
            return mk(cw_ref, cw_ssem, cw_rsem, k_src, k_dst, q, right)

        def mk_ccw(k_src, k_dst, q):
            return mk(ccw_ref, ccw_ssem, ccw_rsem, k_src, k_dst, q, left)

        def mk_ag(t, q, cols, ssem, rsem, dev, g):
            r_src = out_ref.at[pl.ds(g * CHUNK + q * PC, PC), cols]
            return pltpu.make_async_remote_copy(
                src_ref=r_src, dst_ref=r_src,
                send_sem=ssem.at[(t - 1) % N_DEV, q],
                recv_sem=rsem.at[t, q],
                device_id=(dev,), device_id_type=pl.DeviceIdType.MESH)

        cw_ref[0, :, :] = chunk_cw(i)
        ccw_ref[0, :, :] = chunk_ccw(i)
        rs_cw = {}
        rs_ccw = {}
        ag_cw = {}
        ag_ccw = {}
        for q in range(Q):
            rs_cw[(0, q)] = mk_cw(0, 1, q)
            rs_cw[(0, q)].start()
            rs_ccw[(0, q)] = mk_ccw(0, 1, q)
            rs_ccw[(0, q)].start()

        own_cw = lax.rem(i + 1, N_DEV)
        own_ccw = lax.rem(i + N_DEV - 1, N_DEV)

        for k in range(N_DEV - 1):
            c_cw = lax.rem(i + 2 * N_DEV - 1 - k, N_DEV)
            c_ccw = lax.rem(i + 1 + k, N_DEV)
            local_cw = chunk_cw(c_cw)
            local_ccw = chunk_ccw(c_ccw)
            for q in range(Q):
                sl = slice(q * PC, (q + 1) * PC)
                rs_cw[(k, q)].wait()
                if k < N_DEV - 2:
                    cw_ref[k + 1, sl, :] = cw_ref[k + 1, sl, :] + local_cw[sl, :]
                    rs_cw[(k + 1, q)] = mk_cw(k + 1, k + 2, q)
                    rs_cw[(k + 1, q)].start()
                else:
                    g = _gelu(cw_ref[N_DEV - 1, sl, :] + local_cw[sl, :])
                    out_ref[pl.ds(own_cw * CHUNK + q * PC, PC), 0:HALF] = g
                    ag_cw[(0, q)] = mk_ag(0, q, pl.ds(0, HALF), cw_ssem,
                                          cw_rsem, right, own_cw)
                    ag_cw[(0, q)].start()
                rs_ccw[(k, q)].wait()
                if k < N_DEV - 2:
                    ccw_ref[k + 1, sl, :] = ccw_ref[k + 1, sl, :] + local_ccw[sl, :]
                    rs_ccw[(k + 1, q)] = mk_ccw(k + 1, k + 2, q)
                    rs_ccw[(k + 1, q)].start()
                else:
                    g = _gelu(ccw_ref[N_DEV - 1, sl, :] + local_ccw[sl, :])
                    out_ref[pl.ds(own_ccw * CHUNK + q * PC, PC), HALF:N] = g
                    ag_ccw[(0, q)] = mk_ag(0, q, pl.ds(HALF, HALF), ccw_ssem,
                                           ccw_rsem, left, own_ccw)
                    ag_ccw[(0, q)].start()

        for t in range(N_DEV - 1):
            c_cw = lax.rem(i + 2 * N_DEV - t, N_DEV)
            c_ccw = lax.rem(i + t, N_DEV)
            for q in range(Q):
                ag_cw[(t, q)].wait()
                if t < N_DEV - 2:
                    ag_cw[(t + 1, q)] = mk_ag(t + 1, q, pl.ds(0, HALF),
                                              cw_ssem, cw_rsem, right, c_cw)
                    ag_cw[(t + 1, q)].start()
                ag_ccw[(t, q)].wait()
                if t < N_DEV - 2:
                    ag_ccw[(t + 1, q)] = mk_ag(t + 1, q, pl.ds(HALF, HALF),
                                               ccw_ssem, ccw_rsem, left, c_ccw)
                    ag_ccw[(t + 1, q)].start()

    return pl.pallas_call(
        body,
        out_shape=jax.ShapeDtypeStruct((M, N), jnp.float32),
        in_specs=[
            pl.BlockSpec(memory_space=pltpu.VMEM),
            pl.BlockSpec(memory_space=pltpu.VMEM),
        ],
        out_specs=pl.BlockSpec(memory_space=pltpu.VMEM),
        scratch_shapes=[
            pltpu.VMEM((N_DEV, CHUNK, HALF), jnp.float32),
            pltpu.VMEM((N_DEV, CHUNK, HALF), jnp.float32),
            pltpu.SemaphoreType.DMA((N_DEV, Q)),
            pltpu.SemaphoreType.DMA((N_DEV, Q)),
            pltpu.SemaphoreType.DMA((N_DEV, Q)),
            pltpu.SemaphoreType.DMA((N_DEV, Q)),
        ],
        compiler_params=pltpu.CompilerParams(collective_id=0),
    )(x, w_mat)


# device time: 95256 ns/iter; 1.8415x vs baseline; 1.8415x over previous
import jax
import jax.numpy as jnp
from jax import lax
from jax.experimental import pallas as pl
from jax.experimental.pallas import tpu as pltpu

N_DEV = 8
M = 2048
N = 2048
CHUNK = M // N_DEV
HALF = N // 2
Q = 4
PC = CHUNK // Q


def _gelu(y):
    c = 0.7978845608028654
    return 0.5 * y * (1.0 + jnp.tanh(c * (y + 0.044715 * y * y * y)))


def kernel(x, w_mat):
    def body(x_ref, w_ref, out_ref, cw_ref, ccw_ref,
             cw_ssem, cw_rsem, ccw_ssem, ccw_rsem):
        i = lax.axis_index("i")
        left = lax.rem(i + N_DEV - 1, N_DEV)
        right = lax.rem(i + 1, N_DEV)

        barrier = pltpu.get_barrier_semaphore()
        pl.semaphore_signal(barrier, inc=1, device_id=(left,),
                            device_id_type=pl.DeviceIdType.MESH)
        pl.semaphore_signal(barrier, inc=1, device_id=(right,),
                            device_id_type=pl.DeviceIdType.MESH)
        pl.semaphore_wait(barrier, 2)

        def chunk_cw(c):
            xc = x_ref[pl.ds(c * CHUNK, CHUNK), :]
            return jnp.dot(xc, w_ref[:, 0:HALF],
                           preferred_element_type=jnp.float32)

        def chunk_ccw(c):
            xc = x_ref[pl.ds(c * CHUNK, CHUNK), :]
            return jnp.dot(xc, w_ref[:, HALF:N],
                           preferred_element_type=jnp.float32)

        def mk(buf, ssem, rsem, k_src, k_dst, q, dev):
            return pltpu.make_async_remote_copy(
                src_ref=buf.at[k_src, pl.ds(q * PC, PC)],
                dst_ref=buf.at[k_dst, pl.ds(q * PC, PC)],
                send_sem=ssem.at[k_src, q],
                recv_sem=rsem.at[k_dst, q],
                device_id=(dev,), device_id_type=pl.DeviceIdType.MESH)

        def mk_cw(k_src, k_dst, q):
            return mk(cw_ref, cw_ssem, cw_rsem, k_src, k_dst, q, right)

        def mk_ccw(k_src, k_dst, q):
            return mk(ccw_ref, ccw_ssem, ccw_rsem, k_src, k_dst, q, left)

        cw_ref[0, :, :] = chunk_cw(i).astype(jnp.bfloat16)
        ccw_ref[0, :, :] = chunk_ccw(i).astype(jnp.bfloat16)
        rs_cw = {}
        rs_ccw = {}
        ag_cw = {}
        ag_ccw = {}
        for q in range(Q):
            rs_cw[(0, q)] = mk_cw(0, 1, q)
            rs_cw[(0, q)].start()
            rs_ccw[(0, q)] = mk_ccw(0, 1, q)
            rs_ccw[(0, q)].start()

        own_cw = lax.rem(i + 1, N_DEV)
        own_ccw = lax.rem(i + N_DEV - 1, N_DEV)

        for k in range(N_DEV - 1):
            c_cw = lax.rem(i + 2 * N_DEV - 1 - k, N_DEV)
            c_ccw = lax.rem(i + 1 + k, N_DEV)
            local_cw = chunk_cw(c_cw)
            local_ccw = chunk_ccw(c_ccw)
            for q in range(Q):
                sl = slice(q * PC, (q + 1) * PC)
                rs_cw[(k, q)].wait()
                acc_cw = (cw_ref[k + 1, sl, :].astype(jnp.float32)
                          + local_cw[sl, :])
                if k < N_DEV - 2:
                    cw_ref[k + 1, sl, :] = acc_cw.astype(jnp.bfloat16)
                    rs_cw[(k + 1, q)] = mk_cw(k + 1, k + 2, q)
                    rs_cw[(k + 1, q)].start()
                else:
                    g = _gelu(acc_cw)
                    out_ref[pl.ds(own_cw * CHUNK + q * PC, PC), 0:HALF] = g
                    cw_ref[N_DEV - 1, sl, :] = g.astype(jnp.bfloat16)
                    ag_cw[(0, q)] = mk_cw(N_DEV - 1, 0, q)
                    ag_cw[(0, q)].start()
                rs_ccw[(k, q)].wait()
                acc_ccw = (ccw_ref[k + 1, sl, :].astype(jnp.float32)
                           + local_ccw[sl, :])
                if k < N_DEV - 2:
                    ccw_ref[k + 1, sl, :] = acc_ccw.astype(jnp.bfloat16)
                    rs_ccw[(k + 1, q)] = mk_ccw(k + 1, k + 2, q)
                    rs_ccw[(k + 1, q)].start()
                else:
                    g = _gelu(acc_ccw)
                    out_ref[pl.ds(own_ccw * CHUNK + q * PC, PC), HALF:N] = g
                    ccw_ref[N_DEV - 1, sl, :] = g.astype(jnp.bfloat16)
                    ag_ccw[(0, q)] = mk_ccw(N_DEV - 1, 0, q)
                    ag_ccw[(0, q)].start()

        for t in range(N_DEV - 1):
            c_cw = lax.rem(i + 2 * N_DEV - t, N_DEV)
            c_ccw = lax.rem(i + t, N_DEV)
            for q in range(Q):
                sl = slice(q * PC, (q + 1) * PC)
                ag_cw[(t, q)].wait()
                if t < N_DEV - 2:
                    ag_cw[(t + 1, q)] = mk_cw(t, t + 1, q)
                    ag_cw[(t + 1, q)].start()
                out_ref[pl.ds(c_cw * CHUNK + q * PC, PC), 0:HALF] = (
                    cw_ref[t, sl, :].astype(jnp.float32))
                ag_ccw[(t, q)].wait()
                if t < N_DEV - 2:
                    ag_ccw[(t + 1, q)] = mk_ccw(t, t + 1, q)
                    ag_ccw[(t + 1, q)].start()
                out_ref[pl.ds(c_ccw * CHUNK + q * PC, PC), HALF:N] = (
                    ccw_ref[t, sl, :].astype(jnp.float32))

    return pl.pallas_call(
        body,
        out_shape=jax.ShapeDtypeStruct((M, N), jnp.float32),
        in_specs=[
            pl.BlockSpec(memory_space=pltpu.VMEM),
            pl.BlockSpec(memory_space=pltpu.VMEM),
        ],
        out_specs=pl.BlockSpec(memory_space=pltpu.VMEM),
        scratch_shapes=[
            pltpu.VMEM((N_DEV, CHUNK, HALF), jnp.bfloat16),
            pltpu.VMEM((N_DEV, CHUNK, HALF), jnp.bfloat16),
            pltpu.SemaphoreType.DMA((N_DEV, Q)),
            pltpu.SemaphoreType.DMA((N_DEV, Q)),
            pltpu.SemaphoreType.DMA((N_DEV, Q)),
            pltpu.SemaphoreType.DMA((N_DEV, Q)),
        ],
        compiler_params=pltpu.CompilerParams(collective_id=0),
    )(x, w_mat)


# device time: 95010 ns/iter; 1.8462x vs baseline; 1.0026x over previous
import jax
import jax.numpy as jnp
from jax import lax
from jax.experimental import pallas as pl
from jax.experimental.pallas import tpu as pltpu

N_DEV = 8
M = 2048
N = 2048
CHUNK = M // N_DEV
HALF = N // 2
Q = 2
PC = CHUNK // Q


def _gelu(y):
    c = 0.7978845608028654
    return 0.5 * y * (1.0 + jnp.tanh(c * (y + 0.044715 * y * y * y)))


def kernel(x, w_mat):
    def body(x_ref, w_ref, out_ref, cw_ref, ccw_ref,
             cw_ssem, cw_rsem, ccw_ssem, ccw_rsem):
        i = lax.axis_index("i")
        left = lax.rem(i + N_DEV - 1, N_DEV)
        right = lax.rem(i + 1, N_DEV)

        barrier = pltpu.get_barrier_semaphore()
        pl.semaphore_signal(barrier, inc=1, device_id=(left,),
                            device_id_type=pl.DeviceIdType.MESH)
        pl.semaphore_signal(barrier, inc=1, device_id=(right,),
                            device_id_type=pl.DeviceIdType.MESH)
        pl.semaphore_wait(barrier, 2)

        def chunk_cw(c):
            xc = x_ref[pl.ds(c * CHUNK, CHUNK), :]
            return jnp.dot(xc, w_ref[:, 0:HALF],
                           preferred_element_type=jnp.float32)

        def chunk_ccw(c):
            xc = x_ref[pl.ds(c * CHUNK, CHUNK), :]
            return jnp.dot(xc, w_ref[:, HALF:N],
                           preferred_element_type=jnp.float32)

        def mk(buf, ssem, rsem, k_src, k_dst, q, dev):
            return pltpu.make_async_remote_copy(
                src_ref=buf.at[k_src, pl.ds(q * PC, PC)],
                dst_ref=buf.at[k_dst, pl.ds(q * PC, PC)],
                send_sem=ssem.at[k_src, q],
                recv_sem=rsem.at[k_dst, q],
                device_id=(dev,), device_id_type=pl.DeviceIdType.MESH)

        def mk_cw(k_src, k_dst, q):
            return mk(cw_ref, cw_ssem, cw_rsem, k_src, k_dst, q, right)

        def mk_ccw(k_src, k_dst, q):
            return mk(ccw_ref, ccw_ssem, ccw_rsem, k_src, k_dst, q, left)

        cw_ref[0, :, :] = chunk_cw(i).astype(jnp.bfloat16)
        ccw_ref[0, :, :] = chunk_ccw(i).astype(jnp.bfloat16)
        rs_cw = {}
        rs_ccw = {}
        ag_cw = {}
        ag_ccw = {}
        for q in range(Q):
            rs_cw[(0, q)] = mk_cw(0, 1, q)
            rs_cw[(0, q)].start()
            rs_ccw[(0, q)] = mk_ccw(0, 1, q)
            rs_ccw[(0, q)].start()

        own_cw = lax.rem(i + 1, N_DEV)
        own_ccw = lax.rem(i + N_DEV - 1, N_DEV)

        for k in range(N_DEV - 1):
            c_cw = lax.rem(i + 2 * N_DEV - 1 - k, N_DEV)
            c_ccw = lax.rem(i + 1 + k, N_DEV)
            local_cw = chunk_cw(c_cw)
            local_ccw = chunk_ccw(c_ccw)
            for q in range(Q):
                sl = slice(q * PC, (q + 1) * PC)
                rs_cw[(k, q)].wait()
                acc_cw = (cw_ref[k + 1, sl, :].astype(jnp.float32)
                          + local_cw[sl, :])
                if k < N_DEV - 2:
                    cw_ref[k + 1, sl, :] = acc_cw.astype(jnp.bfloat16)
                    rs_cw[(k + 1, q)] = mk_cw(k + 1, k + 2, q)
                    rs_cw[(k + 1, q)].start()
                else:
                    g = _gelu(acc_cw)
                    out_ref[pl.ds(own_cw * CHUNK + q * PC, PC), 0:HALF] = g
                    cw_ref[N_DEV - 1, sl, :] = g.astype(jnp.bfloat16)
                    ag_cw[(0, q)] = mk_cw(N_DEV - 1, 0, q)
                    ag_cw[(0, q)].start()
                rs_ccw[(k, q)].wait()
                acc_ccw = (ccw_ref[k + 1, sl, :].astype(jnp.float32)
                           + local_ccw[sl, :])
                if k < N_DEV - 2:
                    ccw_ref[k + 1, sl, :] = acc_ccw.astype(jnp.bfloat16)
                    rs_ccw[(k + 1, q)] = mk_ccw(k + 1, k + 2, q)
                    rs_ccw[(k + 1, q)].start()
                else:
                    g = _gelu(acc_ccw)
                    out_ref[pl.ds(own_ccw * CHUNK + q * PC, PC), HALF:N] = g
                    ccw_ref[N_DEV - 1, sl, :] = g.astype(jnp.bfloat16)
                    ag_ccw[(0, q)] = mk_ccw(N_DEV - 1, 0, q)
                    ag_ccw[(0, q)].start()

        for t in range(N_DEV - 1):
            c_cw = lax.rem(i + 2 * N_DEV - t, N_DEV)
            c_ccw = lax.rem(i + t, N_DEV)
            for q in range(Q):
                sl = slice(q * PC, (q + 1) * PC)
                ag_cw[(t, q)].wait()
                if t < N_DEV - 2:
                    ag_cw[(t + 1, q)] = mk_cw(t, t + 1, q)
                    ag_cw[(t + 1, q)].start()
                out_ref[pl.ds(c_cw * CHUNK + q * PC, PC), 0:HALF] = (
                    cw_ref[t, sl, :].astype(jnp.float32))
                ag_ccw[(t, q)].wait()
                if t < N_DEV - 2:
                    ag_ccw[(t + 1, q)] = mk_ccw(t, t + 1, q)
                    ag_ccw[(t + 1, q)].start()
                out_ref[pl.ds(c_ccw * CHUNK + q * PC, PC), HALF:N] = (
                    ccw_ref[t, sl, :].astype(jnp.float32))

    return pl.pallas_call(
        body,
        out_shape=jax.ShapeDtypeStruct((M, N), jnp.float32),
        in_specs=[
            pl.BlockSpec(memory_space=pltpu.VMEM),
            pl.BlockSpec(memory_space=pltpu.VMEM),
        ],
        out_specs=pl.BlockSpec(memory_space=pltpu.VMEM),
        scratch_shapes=[
            pltpu.VMEM((N_DEV, CHUNK, HALF), jnp.bfloat16),
            pltpu.VMEM((N_DEV, CHUNK, HALF), jnp.bfloat16),
            pltpu.SemaphoreType.DMA((N_DEV, Q)),
            pltpu.SemaphoreType.DMA((N_DEV, Q)),
            pltpu.SemaphoreType.DMA((N_DEV, Q)),
            pltpu.SemaphoreType.DMA((N_DEV, Q)),
        ],
        compiler_params=pltpu.CompilerParams(collective_id=0),
    )(x, w_mat)
